# baseline (speedup 1.0000x reference)
import jax
import jax.numpy as jnp
from jax.experimental import pallas as pl
from jax.experimental.pallas import tpu as pltpu

B, L, CDL = 32, 64, 128
DIN = 3 * CDL
D_LLM, D_FF, D_MODEL = 4096, 2048, 1024
E, TOP_K = 8, 2
EPS = 1e-9
NT = 4
KT = D_LLM // NT
SB = 8
NB = B // SB


def _fused_kernel(dkp_ref, cyc_ref, w1_ref, w1c_ref, b1_ref, w2_ref, b2_ref,
                  x_ref, we_ref, wg_ref, be_ref, bg_ref, out_ref,
                  acc_ref, web_ref, wgb_ref, xb_ref, yg_ref, vg_ref, sg_ref,
                  sem):
    n = pl.program_id(0)
    dn = (((1,), (0,)), ((), ()))

    @pl.when(n == 0)
    def _():
        wgb_ref[...] = wg_ref[...].astype(jnp.bfloat16)

    @pl.when(n < NT)
    def _():
        eb = E // NT
        web_ref[pl.ds(n * eb, eb)] = we_ref[...].astype(jnp.bfloat16)
        xg = x_ref[...].reshape(SB * L, DIN).astype(jnp.bfloat16)
        xb_ref[pl.ds(n * SB * L, SB * L)] = xg
        yg_ref[pl.ds(n * SB * L, SB * L)] = jax.lax.dot_general(
            xg, wgb_ref[...], dn, preferred_element_type=jnp.float32)

    @pl.when(n < NT)
    def _():
        part = jax.lax.dot_general(dkp_ref[...].astype(jnp.bfloat16),
                                   w1_ref[...].astype(jnp.bfloat16), dn,
                                   preferred_element_type=jnp.float32)

        @pl.when(n == 0)
        def _():
            acc_ref[...] = part

        @pl.when(n > 0)
        def _():
            acc_ref[...] += part

        @pl.when(n == NT - 1)
        def _():
            cyc_b = cyc_ref[...].astype(jnp.bfloat16).astype(jnp.float32)
            w1c_b = w1c_ref[0:1].astype(jnp.bfloat16).astype(jnp.float32)
            h = jnp.maximum(acc_ref[...] + cyc_b * w1c_b + b1_ref[...], 0.0)
            logits = jax.lax.dot_general(h.astype(jnp.bfloat16),
                                         w2_ref[...].astype(jnp.bfloat16), dn,
                                         preferred_element_type=jnp.float32)
            logits = logits + b2_ref[...]
            col = jax.lax.broadcasted_iota(jnp.int32, (B, E), 1)
            m1 = jnp.max(logits, axis=1, keepdims=True)
            e0 = jnp.min(jnp.where(logits == m1, col, E), axis=1)
            oh0 = col == e0[:, None]
            l2 = jnp.where(oh0, -jnp.inf, logits)
            m2 = jnp.max(l2, axis=1, keepdims=True)
            e1 = jnp.min(jnp.where(l2 == m2, col, E), axis=1)
            oh1 = col == e1[:, None]
            ex = jnp.exp(logits - m1)
            probs = ex / jnp.sum(ex, axis=1, keepdims=True)
            p0 = jnp.sum(jnp.where(oh0, probs, 0.0), axis=1)
            p1 = jnp.sum(jnp.where(oh1, probs, 0.0), axis=1)
            den = p0 + p1 + EPS
            packed = jnp.concatenate(
                [e0.astype(jnp.float32)[:, None], e1.astype(jnp.float32)[:, None],
                 (p0 / den)[:, None], (p1 / den)[:, None]], axis=1)
            vg_ref[...] = packed
            pltpu.make_async_copy(vg_ref, sg_ref, sem).start()

    @pl.when(n == NT - 1)
    def _():
        pltpu.make_async_copy(vg_ref, sg_ref, sem).wait()

    @pl.when(n >= NT - 1)
    def _():
        t = n - (NT - 1)
        for s in range(SB):
            b = t * SB + s
            e0 = sg_ref[b, 0].astype(jnp.int32)
            e1 = sg_ref[b, 1].astype(jnp.int32)
            g0 = sg_ref[b, 2]
            g1 = sg_ref[b, 3]
            xs = xb_ref[pl.ds(b * L, L)]
            y0 = jax.lax.dot_general(xs, web_ref[e0], dn,
                                     preferred_element_type=jnp.float32)
            y1 = jax.lax.dot_general(xs, web_ref[e1], dn,
                                     preferred_element_type=jnp.float32)
            out_ref[s] = (yg_ref[pl.ds(b * L, L)] + bg_ref[...]) \
                + g0 * (y0 + be_ref[e0]) + g1 * (y1 + be_ref[e1])


def kernel(cycle_curve_data, cycle_numbers, DKP_embeddings, Wg1, bg1, Wg2, bg2,
           We, be, Wgen, bgen):
    gi = lambda n: (jnp.minimum(n, NT - 1), 0)
    xi = lambda n: (jnp.minimum(n, NT - 1), 0, 0, 0)
    oi = lambda n: (jnp.maximum(n - (NT - 1), 0), 0, 0)

    out = pl.pallas_call(
        _fused_kernel,
        grid=(NT + NB - 1,),
        in_specs=[
            pl.BlockSpec((B, KT), lambda n: (0, jnp.minimum(n, NT - 1))),
            pl.BlockSpec((B, 1), lambda n: (0, 0)),
            pl.BlockSpec((KT, D_FF), gi),
            pl.BlockSpec((8, D_FF), lambda n: (D_LLM // 8, 0)),
            pl.BlockSpec((1, D_FF), lambda n: (0, 0)),
            pl.BlockSpec((D_FF, E), lambda n: (0, 0)),
            pl.BlockSpec((1, E), lambda n: (0, 0)),
            pl.BlockSpec((SB, L, 3, CDL), xi),
            pl.BlockSpec((E // NT, DIN, D_MODEL),
                         lambda n: (jnp.minimum(n, NT - 1), 0, 0)),
            pl.BlockSpec((DIN, D_MODEL), lambda n: (0, 0)),
            pl.BlockSpec((E, D_MODEL), lambda n: (0, 0)),
            pl.BlockSpec((1, D_MODEL), lambda n: (0, 0)),
        ],
        out_specs=pl.BlockSpec((SB, L, D_MODEL), oi),
        out_shape=jax.ShapeDtypeStruct((B, L, D_MODEL), jnp.float32),
        scratch_shapes=[
            pltpu.VMEM((B, D_FF), jnp.float32),
            pltpu.VMEM((E, DIN, D_MODEL), jnp.bfloat16),
            pltpu.VMEM((DIN, D_MODEL), jnp.bfloat16),
            pltpu.VMEM((B * L, DIN), jnp.bfloat16),
            pltpu.VMEM((B * L, D_MODEL), jnp.float32),
            pltpu.VMEM((B, 4), jnp.float32),
            pltpu.SMEM((B, 4), jnp.float32),
            pltpu.SemaphoreType.DMA,
        ],
    )(DKP_embeddings, cycle_numbers, Wg1, Wg1, bg1[None, :], Wg2, bg2[None, :],
      cycle_curve_data, We, Wgen, be, bgen[None, :])

    return out


# speedup vs baseline: 1.0144x; 1.0144x over previous
import jax
import jax.numpy as jnp
from jax.experimental import pallas as pl
from jax.experimental.pallas import tpu as pltpu

B, L, CDL = 32, 64, 128
DIN = 3 * CDL
D_LLM, D_FF, D_MODEL = 4096, 2048, 1024
E, TOP_K = 8, 2
EPS = 1e-9
NT = 4
KT = D_LLM // NT
SB = 8
NB = B // SB


def _fused_kernel(dkp_ref, cyc_ref, w1_ref, w1c_ref, b1_ref, w2_ref, b2_ref,
                  x_ref, we_ref, wg_ref, be_ref, bg_ref, out_ref,
                  acc_ref, web_ref, wgb_ref, xb_ref, yg_ref, vg_ref, sg_ref,
                  sem):
    n = pl.program_id(0)
    dn = (((1,), (0,)), ((), ()))

    @pl.when(n == 0)
    def _():
        wgb_ref[...] = wg_ref[...].astype(jnp.bfloat16)

    @pl.when(n < NT)
    def _():
        eb = E // NT
        web_ref[pl.ds(n * eb, eb)] = we_ref[...].astype(jnp.bfloat16)
        xg = x_ref[...].reshape(SB * L, DIN).astype(jnp.bfloat16)
        xb_ref[pl.ds(n * SB * L, SB * L)] = xg
        yg_ref[pl.ds(n * SB * L, SB * L)] = jax.lax.dot_general(
            xg, wgb_ref[...], dn, preferred_element_type=jnp.float32)

    @pl.when(n < NT)
    def _():
        part = jax.lax.dot_general(dkp_ref[...].astype(jnp.bfloat16),
                                   w1_ref[...].astype(jnp.bfloat16), dn,
                                   preferred_element_type=jnp.float32)

        @pl.when(n == 0)
        def _():
            acc_ref[...] = part

        @pl.when(n > 0)
        def _():
            acc_ref[...] += part

        @pl.when(n == NT - 1)
        def _():
            cyc_b = cyc_ref[...].astype(jnp.bfloat16).astype(jnp.float32)
            w1c_b = w1c_ref[0:1].astype(jnp.bfloat16).astype(jnp.float32)
            h = jnp.maximum(acc_ref[...] + cyc_b * w1c_b + b1_ref[...], 0.0)
            logits = jax.lax.dot_general(h.astype(jnp.bfloat16),
                                         w2_ref[...].astype(jnp.bfloat16), dn,
                                         preferred_element_type=jnp.float32)
            logits = logits + b2_ref[...]
            col = jax.lax.broadcasted_iota(jnp.int32, (B, E), 1)
            m1 = jnp.max(logits, axis=1, keepdims=True)
            e0 = jnp.min(jnp.where(logits == m1, col, E), axis=1)
            oh0 = col == e0[:, None]
            l2 = jnp.where(oh0, -jnp.inf, logits)
            m2 = jnp.max(l2, axis=1, keepdims=True)
            e1 = jnp.min(jnp.where(l2 == m2, col, E), axis=1)
            oh1 = col == e1[:, None]
            ex = jnp.exp(logits - m1)
            probs = ex / jnp.sum(ex, axis=1, keepdims=True)
            p0 = jnp.sum(jnp.where(oh0, probs, 0.0), axis=1)
            p1 = jnp.sum(jnp.where(oh1, probs, 0.0), axis=1)
            den = p0 + p1 + EPS
            packed = jnp.concatenate(
                [e0.astype(jnp.float32)[:, None], e1.astype(jnp.float32)[:, None],
                 (p0 / den)[:, None], (p1 / den)[:, None]], axis=1)
            vg_ref[...] = packed
            pltpu.make_async_copy(vg_ref, sg_ref, sem).start()

    @pl.when(n == NT)
    def _():
        pltpu.make_async_copy(vg_ref, sg_ref, sem).wait()

    @pl.when(n >= NT)
    def _():
        t = n - NT
        for s in range(SB):
            b = t * SB + s
            e0 = sg_ref[b, 0].astype(jnp.int32)
            e1 = sg_ref[b, 1].astype(jnp.int32)
            g0 = sg_ref[b, 2]
            g1 = sg_ref[b, 3]
            xs = xb_ref[pl.ds(b * L, L)]
            y0 = jax.lax.dot_general(xs, web_ref[e0], dn,
                                     preferred_element_type=jnp.float32)
            y1 = jax.lax.dot_general(xs, web_ref[e1], dn,
                                     preferred_element_type=jnp.float32)
            out_ref[s] = (yg_ref[pl.ds(b * L, L)] + bg_ref[...]) \
                + g0 * (y0 + be_ref[e0]) + g1 * (y1 + be_ref[e1])


def kernel(cycle_curve_data, cycle_numbers, DKP_embeddings, Wg1, bg1, Wg2, bg2,
           We, be, Wgen, bgen):
    gi = lambda n: (jnp.minimum(n, NT - 1), 0)
    xi = lambda n: (jnp.minimum(n, NT - 1), 0, 0, 0)
    oi = lambda n: (jnp.maximum(n - NT, 0), 0, 0)

    out = pl.pallas_call(
        _fused_kernel,
        grid=(NT + NB,),
        in_specs=[
            pl.BlockSpec((B, KT), lambda n: (0, jnp.minimum(n, NT - 1))),
            pl.BlockSpec((B, 1), lambda n: (0, 0)),
            pl.BlockSpec((KT, D_FF), gi),
            pl.BlockSpec((8, D_FF), lambda n: (D_LLM // 8, 0)),
            pl.BlockSpec((1, D_FF), lambda n: (0, 0)),
            pl.BlockSpec((D_FF, E), lambda n: (0, 0)),
            pl.BlockSpec((1, E), lambda n: (0, 0)),
            pl.BlockSpec((SB, L, 3, CDL), xi),
            pl.BlockSpec((E // NT, DIN, D_MODEL),
                         lambda n: (jnp.minimum(n, NT - 1), 0, 0)),
            pl.BlockSpec((DIN, D_MODEL), lambda n: (0, 0)),
            pl.BlockSpec((E, D_MODEL), lambda n: (0, 0)),
            pl.BlockSpec((1, D_MODEL), lambda n: (0, 0)),
        ],
        out_specs=pl.BlockSpec((SB, L, D_MODEL), oi),
        out_shape=jax.ShapeDtypeStruct((B, L, D_MODEL), jnp.float32),
        scratch_shapes=[
            pltpu.VMEM((B, D_FF), jnp.float32),
            pltpu.VMEM((E, DIN, D_MODEL), jnp.bfloat16),
            pltpu.VMEM((DIN, D_MODEL), jnp.bfloat16),
            pltpu.VMEM((B * L, DIN), jnp.bfloat16),
            pltpu.VMEM((B * L, D_MODEL), jnp.float32),
            pltpu.VMEM((B, 4), jnp.float32),
            pltpu.SMEM((B, 4), jnp.float32),
            pltpu.SemaphoreType.DMA,
        ],
    )(DKP_embeddings, cycle_numbers, Wg1, Wg1, bg1[None, :], Wg2, bg2[None, :],
      cycle_curve_data, We, Wgen, be, bgen[None, :])

    return out
